# baseline (speedup 1.0000x reference)
import jax
import jax.numpy as jnp
from jax import lax
from jax.experimental import pallas as pl
from jax.experimental.pallas import tpu as pltpu
from jax.experimental.pallas import tpu_sc as plsc

VOCAB = 1_000_000
D = 64
B_TOTAL = 4096 * 200
NC, NS = 2, 16
NW = NC * NS
PER_W = B_TOTAL // NW
CHUNK = 128
NCH = PER_W // CHUNK
NB = 4
SCALE = 8.0


def _body(table_ref, idx_ref, out_ref,
          idxv,
          gb0, gb1, gb2, gb3,
          sb0, sb1, sb2, sb3,
          gs0, gs1, gs2, gs3,
          os0, os1, os2, os3):
  gb = [gb0, gb1, gb2, gb3]
  sb = [sb0, sb1, sb2, sb3]
  gsem = [gs0, gs1, gs2, gs3]
  osem = [os0, os1, os2, os3]

  wid = lax.axis_index("s") * NC + lax.axis_index("c")
  base_row = wid * NCH
  base_out = wid * PER_W

  pltpu.sync_copy(idx_ref.at[pl.ds(base_row, NCH)], idxv)

  for b in range(NB):
    pltpu.async_copy(table_ref.at[idxv.at[b]], gb[b], gsem[b])

  def outer(g, carry):
    for b in range(NB):
      j = g * NB + b
      pltpu.make_async_copy(table_ref.at[idxv.at[j]], gb[b], gsem[b]).wait()

      @pl.when(j >= NB)
      def _():
        pltpu.make_async_copy(
            sb[b], out_ref.at[pl.ds(base_out, CHUNK)], osem[b]).wait()

      def scale_row(r, c2):
        for c in range(D // 16):
          sb[b][r, pl.ds(c * 16, 16)] = gb[b][r, pl.ds(c * 16, 16)] * SCALE
        return c2
      lax.fori_loop(0, CHUNK, scale_row, 0, unroll=4)

      pltpu.async_copy(
          sb[b], out_ref.at[pl.ds(base_out + j * CHUNK, CHUNK)], osem[b])

      @pl.when(j + NB < NCH)
      def _():
        pltpu.async_copy(table_ref.at[idxv.at[j + NB]], gb[b], gsem[b])
    return carry

  lax.fori_loop(0, NCH // NB, outer, 0)

  for b in range(NB):
    pltpu.make_async_copy(
        sb[b], out_ref.at[pl.ds(base_out, CHUNK)], osem[b]).wait()


def kernel(x, table):
  idx = x.astype(jnp.int32).reshape(NW * NCH, CHUNK)
  mesh = plsc.VectorSubcoreMesh(core_axis_name="c", subcore_axis_name="s")
  k = pl.kernel(
      _body,
      mesh=mesh,
      compiler_params=pltpu.CompilerParams(use_tc_tiling_on_sc=False),
      out_type=jax.ShapeDtypeStruct((B_TOTAL, D), jnp.float32),
      scratch_types=(
          [pltpu.VMEM((NCH, CHUNK), jnp.int32)]
          + [pltpu.VMEM((CHUNK, D), jnp.float32) for _ in range(2 * NB)]
          + [pltpu.SemaphoreType.DMA for _ in range(2 * NB)]
      ),
  )
  out = k(table, idx)
  return out.reshape(4096, 200, D)
